# Initial kernel scaffold; baseline (speedup 1.0000x reference)
import functools

import jax
import jax.numpy as jnp
from jax import lax
from jax.experimental import pallas as pl
from jax.experimental.pallas import tpu as pltpu
from jax.experimental.pallas import tpu_sc as plsc

B = 32
NPG = 1516
IN_CH = 128
N = B * NPG
E = 776192
HID = 64
NUM_CLASSES = 10

NTILE = 16
NCORE = 2
N_PAD = 48640
RPT = N_PAD // NTILE
CHUNK = 128
SUP = 8
E_PAD = 786432
G_S = (E_PAD // NTILE) // (CHUNK * SUP)
G_D = (E_PAD // (NTILE * NCORE)) // (CHUNK * SUP)

RB = 3032
GRID_N = N // RB
KB = 3032
GRID_K = (NPG * HID) // KB
EPS = 1e-5

_mesh = plsc.VectorSubcoreMesh(core_axis_name="c", subcore_axis_name="s")


@functools.partial(
    pl.kernel,
    mesh=_mesh,
    out_type=jax.ShapeDtypeStruct((NCORE, N_PAD), jnp.float32),
    scratch_types=[
        pltpu.VMEM_SHARED((N_PAD,), jnp.float32),
        pltpu.VMEM((SUP, CHUNK), jnp.int32),
        pltpu.VMEM((SUP, CHUNK), jnp.float32),
    ],
)
def _sc_deg(dst_hbm, ew_hbm, zero_hbm, out_hbm, acc, dst_v, ew_v):
    cid = lax.axis_index("c")
    sid = lax.axis_index("s")
    pltpu.sync_copy(zero_hbm.at[pl.ds(sid * RPT, RPT)],
                    acc.at[pl.ds(sid * RPT, RPT)])
    plsc.subcore_barrier()

    def sup_body(g, _):
        pltpu.sync_copy(dst_hbm.at[cid, sid, pl.ds(g * SUP, SUP)], dst_v)
        pltpu.sync_copy(ew_hbm.at[cid, sid, pl.ds(g * SUP, SUP)], ew_v)

        def chunk_body(j, _):
            pltpu.sync_copy(ew_v.at[j], acc.at[dst_v.at[j]], add=True)
            return 0

        lax.fori_loop(0, SUP, chunk_body, 0)
        return 0

    lax.fori_loop(0, G_D, sup_body, 0)
    plsc.subcore_barrier()
    pltpu.sync_copy(acc.at[pl.ds(sid * RPT, RPT)],
                    out_hbm.at[cid, pl.ds(sid * RPT, RPT)])


@functools.partial(
    pl.kernel,
    mesh=_mesh,
    out_type=jax.ShapeDtypeStruct((NCORE, N_PAD, 32), jnp.float32),
    scratch_types=[
        pltpu.VMEM_SHARED((N_PAD, 32), jnp.float32),
        pltpu.VMEM((SUP, CHUNK), jnp.int32),
        pltpu.VMEM((SUP, CHUNK), jnp.int32),
        pltpu.VMEM((SUP, CHUNK), jnp.float32),
        pltpu.VMEM((CHUNK, 32), jnp.float32),
    ],
)
def _sc_msg(ytab_hbm, src_hbm, dst_hbm, ew_hbm, zero_hbm, out_hbm,
            acc, src_v, dst_v, ew_v, rows_v):
    cid = lax.axis_index("c")
    sid = lax.axis_index("s")
    pltpu.sync_copy(zero_hbm.at[pl.ds(sid * RPT, RPT)],
                    acc.at[pl.ds(sid * RPT, RPT)])
    plsc.subcore_barrier()

    def sup_body(g, _):
        pltpu.sync_copy(src_hbm.at[cid, sid, pl.ds(g * SUP, SUP)], src_v)
        pltpu.sync_copy(dst_hbm.at[sid, pl.ds(g * SUP, SUP)], dst_v)
        pltpu.sync_copy(ew_hbm.at[sid, pl.ds(g * SUP, SUP)], ew_v)

        def chunk_body(j, _):
            pltpu.sync_copy(ytab_hbm.at[src_v.at[j]], rows_v)

            def scale_body(k, _):
                w = ew_v[j, k]
                rows_v[k, 0:16] = rows_v[k, 0:16] * w
                rows_v[k, 16:32] = rows_v[k, 16:32] * w
                return 0

            lax.fori_loop(0, CHUNK, scale_body, 0)
            pltpu.sync_copy(rows_v, acc.at[dst_v.at[j]], add=True)
            return 0

        lax.fori_loop(0, SUP, chunk_body, 0)
        return 0

    lax.fori_loop(0, G_S, sup_body, 0)
    plsc.subcore_barrier()
    pltpu.sync_copy(acc.at[pl.ds(sid * RPT, RPT)],
                    out_hbm.at[cid, pl.ds(sid * RPT, RPT)])


def _mm1_body(h_ref, w_ref, p0_ref, p1_ref, y_ref, dinv_ref):
    deg = p0_ref[...] + p1_ref[...] + 1.0
    dinv = lax.rsqrt(deg)
    xw = jnp.dot(h_ref[...], w_ref[...], preferred_element_type=jnp.float32)
    y_ref[...] = xw * dinv
    dinv_ref[...] = dinv


def _zsum_body(s0_ref, s1_ref, y_ref, dinv_ref, b_ref, z_ref, sums_ref, acc):
    i = pl.program_id(0)
    s = jnp.concatenate([s0_ref[...], s1_ref[...]], axis=1)
    z = dinv_ref[...] * (s + y_ref[...]) + b_ref[...]
    z_ref[...] = z

    @pl.when(i == 0)
    def _():
        acc[...] = jnp.zeros_like(acc)

    acc[0:1, :] += jnp.sum(z, axis=0, keepdims=True)
    acc[1:2, :] += jnp.sum(z * z, axis=0, keepdims=True)

    @pl.when(i == GRID_N - 1)
    def _():
        sums_ref[...] = acc[...]


def _bn_h(z_ref, sums_ref, g_ref, be_ref):
    mean = sums_ref[0:1, :] * (1.0 / N)
    var = sums_ref[1:2, :] * (1.0 / N) - mean * mean
    inv = lax.rsqrt(var + EPS)
    return jnp.maximum((z_ref[...] - mean) * inv * g_ref[...] + be_ref[...],
                       0.0)


def _bnmm_body(z_ref, sums_ref, g_ref, be_ref, w_ref, dinv_ref, y_ref):
    h = _bn_h(z_ref, sums_ref, g_ref, be_ref)
    y_ref[...] = jnp.dot(h, w_ref[...],
                         preferred_element_type=jnp.float32) * dinv_ref[...]


def _bnfin_body(z_ref, sums_ref, g_ref, be_ref, h_ref):
    h_ref[...] = _bn_h(z_ref, sums_ref, g_ref, be_ref)


def _head_body(hr_ref, wl1_ref, bl1_ref, gl_ref, bel_ref, wl3_ref, bl3_ref,
               out_ref, acc):
    i = pl.program_id(0)

    @pl.when(i == 0)
    def _():
        acc[...] = jnp.zeros_like(acc)

    acc[...] += jnp.dot(hr_ref[...], wl1_ref[...],
                        preferred_element_type=jnp.float32)

    @pl.when(i == GRID_K - 1)
    def _():
        t = acc[...] + bl1_ref[...]
        m = jnp.mean(t, axis=0, keepdims=True)
        v = jnp.mean(t * t, axis=0, keepdims=True) - m * m
        hh = jnp.maximum(
            (t - m) * lax.rsqrt(v + EPS) * gl_ref[...] + bel_ref[...], 0.0)
        out_ref[...] = jnp.dot(hh, wl3_ref[...],
                               preferred_element_type=jnp.float32) + bl3_ref[...]


def _mm1(h0, W1, p0, p1):
    return pl.pallas_call(
        _mm1_body,
        grid=(GRID_N,),
        in_specs=[
            pl.BlockSpec((RB, IN_CH), lambda i: (i, 0)),
            pl.BlockSpec((IN_CH, HID), lambda i: (0, 0)),
            pl.BlockSpec((RB, 1), lambda i: (i, 0)),
            pl.BlockSpec((RB, 1), lambda i: (i, 0)),
        ],
        out_specs=[
            pl.BlockSpec((RB, HID), lambda i: (i, 0)),
            pl.BlockSpec((RB, 1), lambda i: (i, 0)),
        ],
        out_shape=[
            jax.ShapeDtypeStruct((N, HID), jnp.float32),
            jax.ShapeDtypeStruct((N, 1), jnp.float32),
        ],
    )(h0, W1, p0, p1)


def _zsum(s0, s1, y, dinv, b):
    return pl.pallas_call(
        _zsum_body,
        grid=(GRID_N,),
        in_specs=[
            pl.BlockSpec((RB, 32), lambda i: (i, 0)),
            pl.BlockSpec((RB, 32), lambda i: (i, 0)),
            pl.BlockSpec((RB, HID), lambda i: (i, 0)),
            pl.BlockSpec((RB, 1), lambda i: (i, 0)),
            pl.BlockSpec((1, HID), lambda i: (0, 0)),
        ],
        out_specs=[
            pl.BlockSpec((RB, HID), lambda i: (i, 0)),
            pl.BlockSpec((2, HID), lambda i: (0, 0)),
        ],
        out_shape=[
            jax.ShapeDtypeStruct((N, HID), jnp.float32),
            jax.ShapeDtypeStruct((2, HID), jnp.float32),
        ],
        scratch_shapes=[pltpu.VMEM((2, HID), jnp.float32)],
    )(s0, s1, y, dinv, b)


def _bnmm(z, sums, g, be, w, dinv):
    return pl.pallas_call(
        _bnmm_body,
        grid=(GRID_N,),
        in_specs=[
            pl.BlockSpec((RB, HID), lambda i: (i, 0)),
            pl.BlockSpec((2, HID), lambda i: (0, 0)),
            pl.BlockSpec((1, HID), lambda i: (0, 0)),
            pl.BlockSpec((1, HID), lambda i: (0, 0)),
            pl.BlockSpec((HID, HID), lambda i: (0, 0)),
            pl.BlockSpec((RB, 1), lambda i: (i, 0)),
        ],
        out_specs=pl.BlockSpec((RB, HID), lambda i: (i, 0)),
        out_shape=jax.ShapeDtypeStruct((N, HID), jnp.float32),
    )(z, sums, g, be, w, dinv)


def _bnfin(z, sums, g, be):
    return pl.pallas_call(
        _bnfin_body,
        grid=(GRID_N,),
        in_specs=[
            pl.BlockSpec((RB, HID), lambda i: (i, 0)),
            pl.BlockSpec((2, HID), lambda i: (0, 0)),
            pl.BlockSpec((1, HID), lambda i: (0, 0)),
            pl.BlockSpec((1, HID), lambda i: (0, 0)),
        ],
        out_specs=pl.BlockSpec((RB, HID), lambda i: (i, 0)),
        out_shape=jax.ShapeDtypeStruct((N, HID), jnp.float32),
    )(z, sums, g, be)


def _head(hr, WL1, bL1, gL1, beL1, WL3, bL3):
    return pl.pallas_call(
        _head_body,
        grid=(GRID_K,),
        in_specs=[
            pl.BlockSpec((B, KB), lambda i: (0, i)),
            pl.BlockSpec((KB, 128), lambda i: (i, 0)),
            pl.BlockSpec((1, 128), lambda i: (0, 0)),
            pl.BlockSpec((1, 128), lambda i: (0, 0)),
            pl.BlockSpec((1, 128), lambda i: (0, 0)),
            pl.BlockSpec((128, NUM_CLASSES), lambda i: (0, 0)),
            pl.BlockSpec((1, NUM_CLASSES), lambda i: (0, 0)),
        ],
        out_specs=pl.BlockSpec((B, NUM_CLASSES), lambda i: (0, 0)),
        out_shape=jax.ShapeDtypeStruct((B, NUM_CLASSES), jnp.float32),
        scratch_shapes=[pltpu.VMEM((B, 128), jnp.float32)],
    )(hr, WL1, bL1, gL1, beL1, WL3, bL3)


def kernel(x, edge_index, edge_weight, W1, b1, g1, be1, W2, b2, g2, be2,
           W3, b3, g3, be3, WL1, bL1, gL1, beL1, WL3, bL3):
    h0 = jnp.transpose(x, (0, 2, 1)).reshape(N, IN_CH)

    pad = E_PAD - E
    src = jnp.concatenate([edge_index[0], jnp.zeros((pad,), jnp.int32)])
    dst = jnp.concatenate([edge_index[1], jnp.zeros((pad,), jnp.int32)])
    ew = jnp.concatenate([edge_weight, jnp.zeros((pad,), jnp.float32)])
    src_pair = jnp.stack([src, src + N]).reshape(NCORE, NTILE, -1, CHUNK)
    dst_s = dst.reshape(NTILE, -1, CHUNK)
    ew_s = ew.reshape(NTILE, -1, CHUNK)
    dst_d = dst.reshape(NCORE, NTILE, -1, CHUNK)
    ew_d = ew.reshape(NCORE, NTILE, -1, CHUNK)
    zero_row = jnp.zeros((N_PAD, 32), jnp.float32)
    zero_deg = jnp.zeros((N_PAD,), jnp.float32)

    degp = _sc_deg(dst_d, ew_d, zero_deg)
    p0 = degp[0, :N, None]
    p1 = degp[1, :N, None]

    y, dinv = _mm1(h0, W1, p0, p1)
    params = [(b1, g1, be1, W2), (b2, g2, be2, W3), (b3, g3, be3, None)]
    h = None
    for b, g, be, w_next in params:
        ytab = jnp.concatenate([y[:, :32], y[:, 32:]], axis=0)
        s = _sc_msg(ytab, src_pair, dst_s, ew_s, zero_row)
        z, sums = _zsum(s[0, :N], s[1, :N], y, dinv, b[None, :])
        if w_next is not None:
            y = _bnmm(z, sums, g[None, :], be[None, :], w_next, dinv)
        else:
            h = _bnfin(z, sums, g[None, :], be[None, :])

    hr = h.reshape(B, NPG * HID)
    return _head(hr, WL1, bL1[None, :], gL1[None, :], beL1[None, :],
                 WL3, bL3[None, :])


# speedup vs baseline: 8.0553x; 8.0553x over previous
import functools

import jax
import jax.numpy as jnp
from jax import lax
from jax.experimental import pallas as pl
from jax.experimental.pallas import tpu as pltpu
from jax.experimental.pallas import tpu_sc as plsc

B = 32
NPG = 1516
IN_CH = 128
N = B * NPG
E = 776192
HID = 64
NUM_CLASSES = 10

NTILE = 16
NCORE = 2
N_PAD = 48640
RPT = N_PAD // NTILE
CHUNK = 128
SUP = 8
E_PAD = 786432
G_S = (E_PAD // NTILE) // (CHUNK * SUP)
G_D = (E_PAD // (NTILE * NCORE)) // (CHUNK * SUP)

RB = 3032
GRID_N = N // RB
KB = 256
GRID_K = (NPG * HID) // KB
EPS = 1e-5

_mesh = plsc.VectorSubcoreMesh(core_axis_name="c", subcore_axis_name="s")


@functools.partial(
    pl.kernel,
    mesh=_mesh,
    out_type=jax.ShapeDtypeStruct((NCORE * N_PAD,), jnp.float32),
    scratch_types=[
        pltpu.VMEM_SHARED((N_PAD,), jnp.float32),
        pltpu.VMEM((SUP, CHUNK), jnp.int32),
        pltpu.VMEM((SUP, CHUNK), jnp.float32),
        pltpu.VMEM((RPT,), jnp.float32),
    ],
)
def _sc_deg(dst_hbm, ew_hbm, out_hbm, acc, dst_v, ew_v, zb):
    cid = lax.axis_index("c")
    sid = lax.axis_index("s")

    def zb_body(i, _):
        zb[pl.ds(i * 16, 16)] = jnp.zeros((16,), jnp.float32)
        return 0

    lax.fori_loop(0, RPT // 16, zb_body, 0)
    pltpu.sync_copy(zb, acc.at[pl.ds(sid * RPT, RPT)])
    plsc.subcore_barrier()

    def sup_body(g, _):
        pltpu.sync_copy(dst_hbm.at[cid, sid, pl.ds(g * SUP, SUP)], dst_v)
        pltpu.sync_copy(ew_hbm.at[cid, sid, pl.ds(g * SUP, SUP)], ew_v)

        def chunk_body(j, _):
            pltpu.sync_copy(ew_v.at[j], acc.at[dst_v.at[j]], add=True)
            return 0

        lax.fori_loop(0, SUP, chunk_body, 0)
        return 0

    lax.fori_loop(0, G_D, sup_body, 0)
    plsc.subcore_barrier()
    pltpu.sync_copy(acc.at[pl.ds(sid * RPT, RPT)], zb)
    pltpu.sync_copy(zb, out_hbm.at[pl.ds(cid * N_PAD + sid * RPT, RPT)])


@functools.partial(
    pl.kernel,
    mesh=_mesh,
    compiler_params=pltpu.CompilerParams(use_tc_tiling_on_sc=False),
    out_type=jax.ShapeDtypeStruct((NCORE, N_PAD, 32), jnp.float32),
    scratch_types=[
        pltpu.VMEM_SHARED((N_PAD, 32), jnp.float32),
        pltpu.VMEM((SUP, CHUNK), jnp.int32),
        pltpu.VMEM((SUP, CHUNK), jnp.int32),
        pltpu.VMEM((SUP, CHUNK), jnp.float32),
        pltpu.VMEM((CHUNK, 32), jnp.float32),
    ],
)
def _sc_msg(ytab_hbm, src_hbm, dst_hbm, ew_hbm, zero_hbm, out_hbm,
            acc, src_v, dst_v, ew_v, rows_v):
    cid = lax.axis_index("c")
    sid = lax.axis_index("s")
    pltpu.sync_copy(zero_hbm.at[pl.ds(sid * RPT, RPT)],
                    acc.at[pl.ds(sid * RPT, RPT)])
    plsc.subcore_barrier()

    def sup_body(g, _):
        pltpu.sync_copy(src_hbm.at[cid, sid, pl.ds(g * SUP, SUP)], src_v)
        pltpu.sync_copy(dst_hbm.at[sid, pl.ds(g * SUP, SUP)], dst_v)
        pltpu.sync_copy(ew_hbm.at[sid, pl.ds(g * SUP, SUP)], ew_v)

        def chunk_body(j, _):
            pltpu.sync_copy(ytab_hbm.at[src_v.at[j]], rows_v)

            def scale_body(t, _):
                k16 = t * 16
                ew16 = ew_v[j, pl.ds(k16, 16)]
                for i in range(16):
                    w = ew16[i]
                    k = k16 + i
                    rows_v[k, 0:16] = rows_v[k, 0:16] * w
                    rows_v[k, 16:32] = rows_v[k, 16:32] * w
                return 0

            lax.fori_loop(0, CHUNK // 16, scale_body, 0)
            pltpu.sync_copy(rows_v, acc.at[dst_v.at[j]], add=True)
            return 0

        lax.fori_loop(0, SUP, chunk_body, 0)
        return 0

    lax.fori_loop(0, G_S, sup_body, 0)
    plsc.subcore_barrier()
    pltpu.sync_copy(acc.at[pl.ds(sid * RPT, RPT)],
                    out_hbm.at[cid, pl.ds(sid * RPT, RPT)])


def _mm1_body(h_ref, w_ref, p0_ref, p1_ref, y_ref, dinv_ref):
    deg = p0_ref[...] + p1_ref[...] + 1.0
    dinv = lax.rsqrt(deg)
    xw = jnp.dot(h_ref[...], w_ref[...], preferred_element_type=jnp.float32)
    y_ref[...] = xw * dinv
    dinv_ref[...] = dinv


def _zsum_body(s0_ref, s1_ref, y_ref, dinv_ref, b_ref, z_ref, sums_ref, acc):
    i = pl.program_id(0)
    s = jnp.concatenate([s0_ref[...], s1_ref[...]], axis=1)
    z = dinv_ref[...] * (s + y_ref[...]) + b_ref[...]
    z_ref[...] = z

    @pl.when(i == 0)
    def _():
        acc[...] = jnp.zeros_like(acc)

    acc[0:1, :] += jnp.sum(z, axis=0, keepdims=True)
    acc[1:2, :] += jnp.sum(z * z, axis=0, keepdims=True)

    @pl.when(i == GRID_N - 1)
    def _():
        sums_ref[...] = acc[...]


def _bn_h(z_ref, sums_ref, g_ref, be_ref):
    mean = sums_ref[0:1, :] * (1.0 / N)
    var = sums_ref[1:2, :] * (1.0 / N) - mean * mean
    inv = lax.rsqrt(var + EPS)
    return jnp.maximum((z_ref[...] - mean) * inv * g_ref[...] + be_ref[...],
                       0.0)


def _bnmm_body(z_ref, sums_ref, g_ref, be_ref, w_ref, dinv_ref, y_ref):
    h = _bn_h(z_ref, sums_ref, g_ref, be_ref)
    y_ref[...] = jnp.dot(h, w_ref[...],
                         preferred_element_type=jnp.float32) * dinv_ref[...]


def _bnfin_body(z_ref, sums_ref, g_ref, be_ref, h_ref):
    h_ref[...] = _bn_h(z_ref, sums_ref, g_ref, be_ref)


def _head_body(hr_ref, wl1_ref, bl1_ref, gl_ref, bel_ref, wl3_ref, bl3_ref,
               out_ref, acc):
    i = pl.program_id(0)

    @pl.when(i == 0)
    def _():
        acc[...] = jnp.zeros_like(acc)

    acc[...] += jnp.dot(hr_ref[...], wl1_ref[...],
                        preferred_element_type=jnp.float32)

    @pl.when(i == GRID_K - 1)
    def _():
        t = acc[...] + bl1_ref[...]
        m = jnp.mean(t, axis=0, keepdims=True)
        v = jnp.mean(t * t, axis=0, keepdims=True) - m * m
        hh = jnp.maximum(
            (t - m) * lax.rsqrt(v + EPS) * gl_ref[...] + bel_ref[...], 0.0)
        out_ref[...] = jnp.dot(hh, wl3_ref[...],
                               preferred_element_type=jnp.float32) + bl3_ref[...]


def _mm1(h0, W1, p0, p1):
    return pl.pallas_call(
        _mm1_body,
        grid=(GRID_N,),
        in_specs=[
            pl.BlockSpec((RB, IN_CH), lambda i: (i, 0)),
            pl.BlockSpec((IN_CH, HID), lambda i: (0, 0)),
            pl.BlockSpec((RB, 1), lambda i: (i, 0)),
            pl.BlockSpec((RB, 1), lambda i: (i, 0)),
        ],
        out_specs=[
            pl.BlockSpec((RB, HID), lambda i: (i, 0)),
            pl.BlockSpec((RB, 1), lambda i: (i, 0)),
        ],
        out_shape=[
            jax.ShapeDtypeStruct((N, HID), jnp.float32),
            jax.ShapeDtypeStruct((N, 1), jnp.float32),
        ],
    )(h0, W1, p0, p1)


def _zsum(s0, s1, y, dinv, b):
    return pl.pallas_call(
        _zsum_body,
        grid=(GRID_N,),
        in_specs=[
            pl.BlockSpec((RB, 32), lambda i: (i, 0)),
            pl.BlockSpec((RB, 32), lambda i: (i, 0)),
            pl.BlockSpec((RB, HID), lambda i: (i, 0)),
            pl.BlockSpec((RB, 1), lambda i: (i, 0)),
            pl.BlockSpec((1, HID), lambda i: (0, 0)),
        ],
        out_specs=[
            pl.BlockSpec((RB, HID), lambda i: (i, 0)),
            pl.BlockSpec((2, HID), lambda i: (0, 0)),
        ],
        out_shape=[
            jax.ShapeDtypeStruct((N, HID), jnp.float32),
            jax.ShapeDtypeStruct((2, HID), jnp.float32),
        ],
        scratch_shapes=[pltpu.VMEM((2, HID), jnp.float32)],
    )(s0, s1, y, dinv, b)


def _bnmm(z, sums, g, be, w, dinv):
    return pl.pallas_call(
        _bnmm_body,
        grid=(GRID_N,),
        in_specs=[
            pl.BlockSpec((RB, HID), lambda i: (i, 0)),
            pl.BlockSpec((2, HID), lambda i: (0, 0)),
            pl.BlockSpec((1, HID), lambda i: (0, 0)),
            pl.BlockSpec((1, HID), lambda i: (0, 0)),
            pl.BlockSpec((HID, HID), lambda i: (0, 0)),
            pl.BlockSpec((RB, 1), lambda i: (i, 0)),
        ],
        out_specs=pl.BlockSpec((RB, HID), lambda i: (i, 0)),
        out_shape=jax.ShapeDtypeStruct((N, HID), jnp.float32),
    )(z, sums, g, be, w, dinv)


def _bnfin(z, sums, g, be):
    return pl.pallas_call(
        _bnfin_body,
        grid=(GRID_N,),
        in_specs=[
            pl.BlockSpec((RB, HID), lambda i: (i, 0)),
            pl.BlockSpec((2, HID), lambda i: (0, 0)),
            pl.BlockSpec((1, HID), lambda i: (0, 0)),
            pl.BlockSpec((1, HID), lambda i: (0, 0)),
        ],
        out_specs=pl.BlockSpec((RB, HID), lambda i: (i, 0)),
        out_shape=jax.ShapeDtypeStruct((N, HID), jnp.float32),
    )(z, sums, g, be)


def _head(hr, WL1, bL1, gL1, beL1, WL3, bL3):
    return pl.pallas_call(
        _head_body,
        grid=(GRID_K,),
        in_specs=[
            pl.BlockSpec((B, KB), lambda i: (0, i)),
            pl.BlockSpec((KB, 128), lambda i: (i, 0)),
            pl.BlockSpec((1, 128), lambda i: (0, 0)),
            pl.BlockSpec((1, 128), lambda i: (0, 0)),
            pl.BlockSpec((1, 128), lambda i: (0, 0)),
            pl.BlockSpec((128, NUM_CLASSES), lambda i: (0, 0)),
            pl.BlockSpec((1, NUM_CLASSES), lambda i: (0, 0)),
        ],
        out_specs=pl.BlockSpec((B, NUM_CLASSES), lambda i: (0, 0)),
        out_shape=jax.ShapeDtypeStruct((B, NUM_CLASSES), jnp.float32),
        scratch_shapes=[pltpu.VMEM((B, 128), jnp.float32)],
    )(hr, WL1, bL1, gL1, beL1, WL3, bL3)


def kernel(x, edge_index, edge_weight, W1, b1, g1, be1, W2, b2, g2, be2,
           W3, b3, g3, be3, WL1, bL1, gL1, beL1, WL3, bL3):
    h0 = jnp.transpose(x, (0, 2, 1)).reshape(N, IN_CH)

    pad = E_PAD - E
    src = jnp.concatenate([edge_index[0], jnp.zeros((pad,), jnp.int32)])
    dst = jnp.concatenate([edge_index[1], jnp.zeros((pad,), jnp.int32)])
    ew = jnp.concatenate([edge_weight, jnp.zeros((pad,), jnp.float32)])
    src_pair = jnp.stack([src, src + N]).reshape(NCORE, NTILE, -1, CHUNK)
    dst_s = dst.reshape(NTILE, -1, CHUNK)
    ew_s = ew.reshape(NTILE, -1, CHUNK)
    dst_d = dst.reshape(NCORE, NTILE, -1, CHUNK)
    ew_d = ew.reshape(NCORE, NTILE, -1, CHUNK)
    zero_row = jnp.zeros((N_PAD, 32), jnp.float32)

    degp = _sc_deg(dst_d, ew_d)
    p0 = degp[:N, None]
    p1 = degp[N_PAD:N_PAD + N, None]

    y, dinv = _mm1(h0, W1, p0, p1)
    params = [(b1, g1, be1, W2), (b2, g2, be2, W3), (b3, g3, be3, None)]
    h = None
    for b, g, be, w_next in params:
        ytab = jnp.concatenate([y[:, :32], y[:, 32:]], axis=0)
        s = _sc_msg(ytab, src_pair, dst_s, ew_s, zero_row)
        z, sums = _zsum(s[0, :N], s[1, :N], y, dinv, b[None, :])
        if w_next is not None:
            y = _bnmm(z, sums, g[None, :], be[None, :], w_next, dinv)
        else:
            h = _bnfin(z, sums, g[None, :], be[None, :])

    hr = h.reshape(B, NPG * HID)
    return _head(hr, WL1, bL1[None, :], gL1[None, :], beL1[None, :],
                 WL3, bL3[None, :])


# speedup vs baseline: 11.9176x; 1.4795x over previous
import functools

import jax
import jax.numpy as jnp
from jax import lax
from jax.experimental import pallas as pl
from jax.experimental.pallas import tpu as pltpu
from jax.experimental.pallas import tpu_sc as plsc

B = 32
NPG = 1516
IN_CH = 128
N = B * NPG
E = 776192
HID = 64
NUM_CLASSES = 10

NTILE = 16
NCORE = 2
N_PAD = 48640
RPT = N_PAD // NTILE
CHUNK = 128
SUP = 8
MSUP = 8
NBUF = 4
LAG = 2
E_PAD = 786432
CPT = (E_PAD // NTILE) // CHUNK
G_S = CPT // MSUP
G_D = (E_PAD // (NTILE * NCORE)) // (CHUNK * SUP)

RB = 3032
GRID_N = N // RB
KB = 256
GRID_K = (NPG * HID) // KB
EPS = 1e-5

_mesh = plsc.VectorSubcoreMesh(core_axis_name="c", subcore_axis_name="s")


@functools.partial(
    pl.kernel,
    mesh=_mesh,
    out_type=jax.ShapeDtypeStruct((NCORE * N_PAD,), jnp.float32),
    scratch_types=[
        pltpu.VMEM_SHARED((N_PAD,), jnp.float32),
        pltpu.VMEM((SUP, CHUNK), jnp.int32),
        pltpu.VMEM((SUP, CHUNK), jnp.float32),
        pltpu.VMEM((RPT,), jnp.float32),
    ],
)
def _sc_deg(dst_hbm, ew_hbm, out_hbm, acc, dst_v, ew_v, zb):
    cid = lax.axis_index("c")
    sid = lax.axis_index("s")

    def zb_body(i, _):
        zb[pl.ds(i * 16, 16)] = jnp.zeros((16,), jnp.float32)
        return 0

    lax.fori_loop(0, RPT // 16, zb_body, 0)
    pltpu.sync_copy(zb, acc.at[pl.ds(sid * RPT, RPT)])
    plsc.subcore_barrier()

    def sup_body(g, _):
        pltpu.sync_copy(dst_hbm.at[cid, sid, pl.ds(g * SUP, SUP)], dst_v)
        pltpu.sync_copy(ew_hbm.at[cid, sid, pl.ds(g * SUP, SUP)], ew_v)

        def chunk_body(j, _):
            pltpu.sync_copy(ew_v.at[j], acc.at[dst_v.at[j]], add=True)
            return 0

        lax.fori_loop(0, SUP, chunk_body, 0)
        return 0

    lax.fori_loop(0, G_D, sup_body, 0)
    plsc.subcore_barrier()
    pltpu.sync_copy(acc.at[pl.ds(sid * RPT, RPT)], zb)
    pltpu.sync_copy(zb, out_hbm.at[pl.ds(cid * N_PAD + sid * RPT, RPT)])


@functools.partial(
    pl.kernel,
    mesh=_mesh,
    compiler_params=pltpu.CompilerParams(use_tc_tiling_on_sc=False),
    out_type=jax.ShapeDtypeStruct((NCORE, N_PAD, 32), jnp.float32),
    scratch_types=[
        pltpu.VMEM_SHARED((N_PAD, 32), jnp.float32),
        pltpu.VMEM((2, MSUP, CHUNK), jnp.int32),
        pltpu.VMEM((2, MSUP, CHUNK), jnp.int32),
        pltpu.VMEM((2, MSUP, CHUNK), jnp.float32),
        pltpu.VMEM((NBUF, CHUNK, 32), jnp.float32),
    ] + [pltpu.SemaphoreType.DMA] * (2 * NBUF + 2),
)
def _sc_msg(ytab_hbm, src_hbm, dst_hbm, ew_hbm, zero_hbm, out_hbm,
            acc, src_v, dst_v, ew_v, rows, *sems):
    gsem = sems[:NBUF]
    ssem = sems[NBUF:2 * NBUF]
    isem = sems[2 * NBUF:]
    cid = lax.axis_index("c")
    sid = lax.axis_index("s")
    pltpu.sync_copy(zero_hbm.at[pl.ds(sid * RPT, RPT)],
                    acc.at[pl.ds(sid * RPT, RPT)])
    plsc.subcore_barrier()

    def fire_idx(g, p):
        return (
            pltpu.async_copy(src_hbm.at[cid, sid, pl.ds(g * MSUP, MSUP)],
                             src_v.at[p], isem[p]),
            pltpu.async_copy(dst_hbm.at[sid, pl.ds(g * MSUP, MSUP)],
                             dst_v.at[p], isem[p]),
            pltpu.async_copy(ew_hbm.at[sid, pl.ds(g * MSUP, MSUP)],
                             ew_v.at[p], isem[p]),
        )

    def drain_idx(g, p):
        pltpu.make_async_copy(src_hbm.at[cid, sid, pl.ds(g * MSUP, MSUP)],
                              src_v.at[p], isem[p]).wait()
        pltpu.make_async_copy(dst_hbm.at[sid, pl.ds(g * MSUP, MSUP)],
                              dst_v.at[p], isem[p]).wait()
        pltpu.make_async_copy(ew_hbm.at[sid, pl.ds(g * MSUP, MSUP)],
                              ew_v.at[p], isem[p]).wait()

    fire_idx(0, 0)

    def do_scale(p, j):
        b = j % NBUF

        def scale_t(t, _):
            k16 = t * 16
            ew16 = ew_v[p, j, pl.ds(k16, 16)]
            for i in range(16):
                w = ew16[i]
                k = k16 + i
                rows[b, k, 0:16] = rows[b, k, 0:16] * w
                rows[b, k, 16:32] = rows[b, k, 16:32] * w
            return 0

        lax.fori_loop(0, CHUNK // 16, scale_t, 0)

    def pair_body(q, _):
        for p in (0, 1):
            g = 2 * q + p
            drain_idx(g, p)

            @pl.when(g + 1 < G_S)
            def _():
                fire_idx(g + 1, 1 - p)

            gat = [None] * MSUP
            scat = [None] * MSUP

            def stage2(jj):
                bb = jj % NBUF
                gat[jj].wait()
                do_scale(p, jj)
                scat[jj] = pltpu.async_copy(
                    rows.at[bb], acc.at[dst_v.at[p, jj]], ssem[bb], add=True)

            for j in range(MSUP):
                b = j % NBUF
                if j >= NBUF:
                    scat[j - NBUF].wait()
                gat[j] = pltpu.async_copy(ytab_hbm.at[src_v.at[p, j]],
                                          rows.at[b], gsem[b])
                if j >= LAG:
                    stage2(j - LAG)
            for jj in range(MSUP - LAG, MSUP):
                stage2(jj)
            for jj in range(MSUP - NBUF, MSUP):
                scat[jj].wait()
        return 0

    lax.fori_loop(0, G_S // 2, pair_body, 0)
    plsc.subcore_barrier()
    pltpu.sync_copy(acc.at[pl.ds(sid * RPT, RPT)],
                    out_hbm.at[cid, pl.ds(sid * RPT, RPT)])


def _mm1_body(h_ref, w_ref, p0_ref, p1_ref, y_ref, dinv_ref):
    deg = p0_ref[...] + p1_ref[...] + 1.0
    dinv = lax.rsqrt(deg)
    xw = jnp.dot(h_ref[...], w_ref[...], preferred_element_type=jnp.float32)
    y_ref[...] = xw * dinv
    dinv_ref[...] = dinv


def _zsum_body(s0_ref, s1_ref, y_ref, dinv_ref, b_ref, z_ref, sums_ref, acc):
    i = pl.program_id(0)
    s = jnp.concatenate([s0_ref[...], s1_ref[...]], axis=1)
    z = dinv_ref[...] * (s + y_ref[...]) + b_ref[...]
    z_ref[...] = z

    @pl.when(i == 0)
    def _():
        acc[...] = jnp.zeros_like(acc)

    acc[0:1, :] += jnp.sum(z, axis=0, keepdims=True)
    acc[1:2, :] += jnp.sum(z * z, axis=0, keepdims=True)

    @pl.when(i == GRID_N - 1)
    def _():
        sums_ref[...] = acc[...]


def _bn_h(z_ref, sums_ref, g_ref, be_ref):
    mean = sums_ref[0:1, :] * (1.0 / N)
    var = sums_ref[1:2, :] * (1.0 / N) - mean * mean
    inv = lax.rsqrt(var + EPS)
    return jnp.maximum((z_ref[...] - mean) * inv * g_ref[...] + be_ref[...],
                       0.0)


def _bnmm_body(z_ref, sums_ref, g_ref, be_ref, w_ref, dinv_ref, y_ref):
    h = _bn_h(z_ref, sums_ref, g_ref, be_ref)
    y_ref[...] = jnp.dot(h, w_ref[...],
                         preferred_element_type=jnp.float32) * dinv_ref[...]


def _bnfin_body(z_ref, sums_ref, g_ref, be_ref, h_ref):
    h_ref[...] = _bn_h(z_ref, sums_ref, g_ref, be_ref)


def _head_body(hr_ref, wl1_ref, bl1_ref, gl_ref, bel_ref, wl3_ref, bl3_ref,
               out_ref, acc):
    i = pl.program_id(0)

    @pl.when(i == 0)
    def _():
        acc[...] = jnp.zeros_like(acc)

    acc[...] += jnp.dot(hr_ref[...], wl1_ref[...],
                        preferred_element_type=jnp.float32)

    @pl.when(i == GRID_K - 1)
    def _():
        t = acc[...] + bl1_ref[...]
        m = jnp.mean(t, axis=0, keepdims=True)
        v = jnp.mean(t * t, axis=0, keepdims=True) - m * m
        hh = jnp.maximum(
            (t - m) * lax.rsqrt(v + EPS) * gl_ref[...] + bel_ref[...], 0.0)
        out_ref[...] = jnp.dot(hh, wl3_ref[...],
                               preferred_element_type=jnp.float32) + bl3_ref[...]


def _mm1(h0, W1, p0, p1):
    return pl.pallas_call(
        _mm1_body,
        grid=(GRID_N,),
        in_specs=[
            pl.BlockSpec((RB, IN_CH), lambda i: (i, 0)),
            pl.BlockSpec((IN_CH, HID), lambda i: (0, 0)),
            pl.BlockSpec((RB, 1), lambda i: (i, 0)),
            pl.BlockSpec((RB, 1), lambda i: (i, 0)),
        ],
        out_specs=[
            pl.BlockSpec((RB, HID), lambda i: (i, 0)),
            pl.BlockSpec((RB, 1), lambda i: (i, 0)),
        ],
        out_shape=[
            jax.ShapeDtypeStruct((N, HID), jnp.float32),
            jax.ShapeDtypeStruct((N, 1), jnp.float32),
        ],
    )(h0, W1, p0, p1)


def _zsum(s0, s1, y, dinv, b):
    return pl.pallas_call(
        _zsum_body,
        grid=(GRID_N,),
        in_specs=[
            pl.BlockSpec((RB, 32), lambda i: (i, 0)),
            pl.BlockSpec((RB, 32), lambda i: (i, 0)),
            pl.BlockSpec((RB, HID), lambda i: (i, 0)),
            pl.BlockSpec((RB, 1), lambda i: (i, 0)),
            pl.BlockSpec((1, HID), lambda i: (0, 0)),
        ],
        out_specs=[
            pl.BlockSpec((RB, HID), lambda i: (i, 0)),
            pl.BlockSpec((2, HID), lambda i: (0, 0)),
        ],
        out_shape=[
            jax.ShapeDtypeStruct((N, HID), jnp.float32),
            jax.ShapeDtypeStruct((2, HID), jnp.float32),
        ],
        scratch_shapes=[pltpu.VMEM((2, HID), jnp.float32)],
    )(s0, s1, y, dinv, b)


def _bnmm(z, sums, g, be, w, dinv):
    return pl.pallas_call(
        _bnmm_body,
        grid=(GRID_N,),
        in_specs=[
            pl.BlockSpec((RB, HID), lambda i: (i, 0)),
            pl.BlockSpec((2, HID), lambda i: (0, 0)),
            pl.BlockSpec((1, HID), lambda i: (0, 0)),
            pl.BlockSpec((1, HID), lambda i: (0, 0)),
            pl.BlockSpec((HID, HID), lambda i: (0, 0)),
            pl.BlockSpec((RB, 1), lambda i: (i, 0)),
        ],
        out_specs=pl.BlockSpec((RB, HID), lambda i: (i, 0)),
        out_shape=jax.ShapeDtypeStruct((N, HID), jnp.float32),
    )(z, sums, g, be, w, dinv)


def _bnfin(z, sums, g, be):
    return pl.pallas_call(
        _bnfin_body,
        grid=(GRID_N,),
        in_specs=[
            pl.BlockSpec((RB, HID), lambda i: (i, 0)),
            pl.BlockSpec((2, HID), lambda i: (0, 0)),
            pl.BlockSpec((1, HID), lambda i: (0, 0)),
            pl.BlockSpec((1, HID), lambda i: (0, 0)),
        ],
        out_specs=pl.BlockSpec((RB, HID), lambda i: (i, 0)),
        out_shape=jax.ShapeDtypeStruct((N, HID), jnp.float32),
    )(z, sums, g, be)


def _head(hr, WL1, bL1, gL1, beL1, WL3, bL3):
    return pl.pallas_call(
        _head_body,
        grid=(GRID_K,),
        in_specs=[
            pl.BlockSpec((B, KB), lambda i: (0, i)),
            pl.BlockSpec((KB, 128), lambda i: (i, 0)),
            pl.BlockSpec((1, 128), lambda i: (0, 0)),
            pl.BlockSpec((1, 128), lambda i: (0, 0)),
            pl.BlockSpec((1, 128), lambda i: (0, 0)),
            pl.BlockSpec((128, NUM_CLASSES), lambda i: (0, 0)),
            pl.BlockSpec((1, NUM_CLASSES), lambda i: (0, 0)),
        ],
        out_specs=pl.BlockSpec((B, NUM_CLASSES), lambda i: (0, 0)),
        out_shape=jax.ShapeDtypeStruct((B, NUM_CLASSES), jnp.float32),
        scratch_shapes=[pltpu.VMEM((B, 128), jnp.float32)],
    )(hr, WL1, bL1, gL1, beL1, WL3, bL3)


def kernel(x, edge_index, edge_weight, W1, b1, g1, be1, W2, b2, g2, be2,
           W3, b3, g3, be3, WL1, bL1, gL1, beL1, WL3, bL3):
    h0 = jnp.transpose(x, (0, 2, 1)).reshape(N, IN_CH)

    pad = E_PAD - E
    src = jnp.concatenate([edge_index[0], jnp.zeros((pad,), jnp.int32)])
    dst = jnp.concatenate([edge_index[1], jnp.zeros((pad,), jnp.int32)])
    ew = jnp.concatenate([edge_weight, jnp.zeros((pad,), jnp.float32)])
    src_pair = jnp.stack([src, src + N]).reshape(NCORE, NTILE, -1, CHUNK)
    dst_s = dst.reshape(NTILE, -1, CHUNK)
    ew_s = ew.reshape(NTILE, -1, CHUNK)
    dst_d = dst.reshape(NCORE, NTILE, -1, CHUNK)
    ew_d = ew.reshape(NCORE, NTILE, -1, CHUNK)
    zero_row = jnp.zeros((N_PAD, 32), jnp.float32)

    degp = _sc_deg(dst_d, ew_d)
    p0 = degp[:N, None]
    p1 = degp[N_PAD:N_PAD + N, None]

    y, dinv = _mm1(h0, W1, p0, p1)
    params = [(b1, g1, be1, W2), (b2, g2, be2, W3), (b3, g3, be3, None)]
    h = None
    for b, g, be, w_next in params:
        ytab = jnp.concatenate([y[:, :32], y[:, 32:]], axis=0)
        s = _sc_msg(ytab, src_pair, dst_s, ew_s, zero_row)
        z, sums = _zsum(s[0, :N], s[1, :N], y, dinv, b[None, :])
        if w_next is not None:
            y = _bnmm(z, sums, g[None, :], be[None, :], w_next, dinv)
        else:
            h = _bnfin(z, sums, g[None, :], be[None, :])

    hr = h.reshape(B, NPG * HID)
    return _head(hr, WL1, bL1[None, :], gL1[None, :], beL1[None, :],
                 WL3, bL3[None, :])


# speedup vs baseline: 12.5637x; 1.0542x over previous
import functools

import jax
import jax.numpy as jnp
from jax import lax
from jax.experimental import pallas as pl
from jax.experimental.pallas import tpu as pltpu
from jax.experimental.pallas import tpu_sc as plsc

B = 32
NPG = 1516
IN_CH = 128
N = B * NPG
E = 776192
HID = 64
NUM_CLASSES = 10

NTILE = 16
NCORE = 2
N_PAD = 48640
RPT = N_PAD // NTILE
CHUNK = 128
SUP = 8
MSUP = 8
NBUF = 4
LAG = 2
E_PAD = 786432
CPT = (E_PAD // NTILE) // CHUNK
G_S = CPT // MSUP
G_D = (E_PAD // (NTILE * NCORE)) // (CHUNK * SUP)

RB = 3032
GRID_N = N // RB
KB = 256
GRID_K = (NPG * HID) // KB
EPS = 1e-5

_mesh = plsc.VectorSubcoreMesh(core_axis_name="c", subcore_axis_name="s")


@functools.partial(
    pl.kernel,
    mesh=_mesh,
    out_type=jax.ShapeDtypeStruct((NCORE * N_PAD,), jnp.float32),
    scratch_types=[
        pltpu.VMEM_SHARED((N_PAD,), jnp.float32),
        pltpu.VMEM((SUP, CHUNK), jnp.int32),
        pltpu.VMEM((SUP, CHUNK), jnp.float32),
        pltpu.VMEM((RPT,), jnp.float32),
    ],
)
def _sc_deg(dst_hbm, ew_hbm, out_hbm, acc, dst_v, ew_v, zb):
    cid = lax.axis_index("c")
    sid = lax.axis_index("s")

    def zb_body(i, _):
        zb[pl.ds(i * 16, 16)] = jnp.zeros((16,), jnp.float32)
        return 0

    lax.fori_loop(0, RPT // 16, zb_body, 0)
    pltpu.sync_copy(zb, acc.at[pl.ds(sid * RPT, RPT)])
    plsc.subcore_barrier()

    def sup_body(g, _):
        pltpu.sync_copy(dst_hbm.at[cid, sid, pl.ds(g * SUP, SUP)], dst_v)
        pltpu.sync_copy(ew_hbm.at[cid, sid, pl.ds(g * SUP, SUP)], ew_v)

        def chunk_body(j, _):
            pltpu.sync_copy(ew_v.at[j], acc.at[dst_v.at[j]], add=True)
            return 0

        lax.fori_loop(0, SUP, chunk_body, 0)
        return 0

    lax.fori_loop(0, G_D, sup_body, 0)
    plsc.subcore_barrier()
    pltpu.sync_copy(acc.at[pl.ds(sid * RPT, RPT)], zb)
    pltpu.sync_copy(zb, out_hbm.at[pl.ds(cid * N_PAD + sid * RPT, RPT)])


@functools.partial(
    pl.kernel,
    mesh=_mesh,
    compiler_params=pltpu.CompilerParams(use_tc_tiling_on_sc=False),
    out_type=jax.ShapeDtypeStruct((4, N_PAD, 16), jnp.float32),
    scratch_types=[
        pltpu.VMEM_SHARED((N_PAD, 16), jnp.float32),
        pltpu.VMEM_SHARED((N_PAD, 16), jnp.float32),
        pltpu.VMEM((2, MSUP, CHUNK), jnp.int32),
        pltpu.VMEM((2, MSUP, CHUNK), jnp.int32),
        pltpu.VMEM((2, MSUP, CHUNK), jnp.float32),
        pltpu.VMEM((NBUF, CHUNK, 16), jnp.float32),
    ] + [pltpu.SemaphoreType.DMA] * (2 * NBUF + 2),
)
def _sc_msg(yq_hbm, src_hbm, dst_hbm, ew_hbm, zero_hbm, out_hbm,
            acc, table, src_v, dst_v, ew_v, rows, *sems):
    gsem = sems[:NBUF]
    ssem = sems[NBUF:2 * NBUF]
    isem = sems[2 * NBUF:]
    cid = lax.axis_index("c")
    sid = lax.axis_index("s")

    def fire_idx(g, p):
        pltpu.async_copy(src_hbm.at[sid, pl.ds(g * MSUP, MSUP)],
                         src_v.at[p], isem[p])
        pltpu.async_copy(dst_hbm.at[sid, pl.ds(g * MSUP, MSUP)],
                         dst_v.at[p], isem[p])
        pltpu.async_copy(ew_hbm.at[sid, pl.ds(g * MSUP, MSUP)],
                         ew_v.at[p], isem[p])

    def drain_idx(g, p):
        pltpu.make_async_copy(src_hbm.at[sid, pl.ds(g * MSUP, MSUP)],
                              src_v.at[p], isem[p]).wait()
        pltpu.make_async_copy(dst_hbm.at[sid, pl.ds(g * MSUP, MSUP)],
                              dst_v.at[p], isem[p]).wait()
        pltpu.make_async_copy(ew_hbm.at[sid, pl.ds(g * MSUP, MSUP)],
                              ew_v.at[p], isem[p]).wait()

    def do_scale(p, j):
        b = j % NBUF

        def scale_t(t, _):
            k16 = t * 16
            ew16 = ew_v[p, j, pl.ds(k16, 16)]
            for i in range(16):
                w = ew16[i]
                k = k16 + i
                rows[b, k, 0:16] = rows[b, k, 0:16] * w
            return 0

        lax.fori_loop(0, CHUNK // 16, scale_t, 0)

    def pair_body(q, _):
        for p in (0, 1):
            g = 2 * q + p
            drain_idx(g, p)

            @pl.when(g + 1 < G_S)
            def _():
                fire_idx(g + 1, 1 - p)

            gat = [None] * MSUP
            scat = [None] * MSUP

            def stage2(jj):
                bb = jj % NBUF
                gat[jj].wait()
                do_scale(p, jj)
                scat[jj] = pltpu.async_copy(
                    rows.at[bb], acc.at[dst_v.at[p, jj]], ssem[bb], add=True)

            for j in range(MSUP):
                b = j % NBUF
                if j >= NBUF:
                    scat[j - NBUF].wait()
                gat[j] = pltpu.async_copy(table.at[src_v.at[p, j]],
                                          rows.at[b], gsem[b])
                if j >= LAG:
                    stage2(j - LAG)
            for jj in range(MSUP - LAG, MSUP):
                stage2(jj)
            for jj in range(MSUP - NBUF, MSUP):
                scat[jj].wait()
        return 0

    for t in range(2):
        qq = cid * 2 + t
        pltpu.sync_copy(yq_hbm.at[qq, pl.ds(sid * RPT, RPT)],
                        table.at[pl.ds(sid * RPT, RPT)])
        pltpu.sync_copy(zero_hbm.at[pl.ds(sid * RPT, RPT)],
                        acc.at[pl.ds(sid * RPT, RPT)])
        plsc.subcore_barrier()
        fire_idx(0, 0)
        lax.fori_loop(0, G_S // 2, pair_body, 0)
        plsc.subcore_barrier()
        pltpu.sync_copy(acc.at[pl.ds(sid * RPT, RPT)],
                        out_hbm.at[qq, pl.ds(sid * RPT, RPT)])


def _mm1_body(h_ref, w_ref, p0_ref, p1_ref, y_ref, dinv_ref):
    deg = p0_ref[...] + p1_ref[...] + 1.0
    dinv = lax.rsqrt(deg)
    xw = jnp.dot(h_ref[...], w_ref[0], preferred_element_type=jnp.float32)
    y_ref[0] = xw * dinv
    dinv_ref[...] = dinv


def _zsum_body(s_ref, y_ref, dinv_ref, b_ref, z_ref, sums_ref, acc):
    i = pl.program_id(0)
    sy = s_ref[...] + y_ref[...]
    z = dinv_ref[...] * jnp.concatenate([sy[q] for q in range(4)],
                                        axis=1) + b_ref[...]
    z_ref[...] = z

    @pl.when(i == 0)
    def _():
        acc[...] = jnp.zeros_like(acc)

    acc[0:1, :] += jnp.sum(z, axis=0, keepdims=True)
    acc[1:2, :] += jnp.sum(z * z, axis=0, keepdims=True)

    @pl.when(i == GRID_N - 1)
    def _():
        sums_ref[...] = acc[...]


def _bn_h(z_ref, sums_ref, g_ref, be_ref):
    mean = sums_ref[0:1, :] * (1.0 / N)
    var = sums_ref[1:2, :] * (1.0 / N) - mean * mean
    inv = lax.rsqrt(var + EPS)
    return jnp.maximum((z_ref[...] - mean) * inv * g_ref[...] + be_ref[...],
                       0.0)


def _bnmm_body(z_ref, sums_ref, g_ref, be_ref, w_ref, dinv_ref, y_ref):
    h = _bn_h(z_ref, sums_ref, g_ref, be_ref)
    y_ref[0] = jnp.dot(h, w_ref[0],
                       preferred_element_type=jnp.float32) * dinv_ref[...]


def _bnfin_body(z_ref, sums_ref, g_ref, be_ref, h_ref):
    h_ref[...] = _bn_h(z_ref, sums_ref, g_ref, be_ref)


def _head_body(hr_ref, wl1_ref, bl1_ref, gl_ref, bel_ref, wl3_ref, bl3_ref,
               out_ref, acc):
    i = pl.program_id(0)

    @pl.when(i == 0)
    def _():
        acc[...] = jnp.zeros_like(acc)

    acc[...] += jnp.dot(hr_ref[...], wl1_ref[...],
                        preferred_element_type=jnp.float32)

    @pl.when(i == GRID_K - 1)
    def _():
        t = acc[...] + bl1_ref[...]
        m = jnp.mean(t, axis=0, keepdims=True)
        v = jnp.mean(t * t, axis=0, keepdims=True) - m * m
        hh = jnp.maximum(
            (t - m) * lax.rsqrt(v + EPS) * gl_ref[...] + bel_ref[...], 0.0)
        out_ref[...] = jnp.dot(hh, wl3_ref[...],
                               preferred_element_type=jnp.float32) + bl3_ref[...]


def _mm1(h0, W1, p0, p1):
    return pl.pallas_call(
        _mm1_body,
        grid=(GRID_N, 4),
        in_specs=[
            pl.BlockSpec((RB, IN_CH), lambda i, j: (i, 0)),
            pl.BlockSpec((1, IN_CH, 16), lambda i, j: (j, 0, 0)),
            pl.BlockSpec((RB, 1), lambda i, j: (i, 0)),
            pl.BlockSpec((RB, 1), lambda i, j: (i, 0)),
        ],
        out_specs=[
            pl.BlockSpec((1, RB, 16), lambda i, j: (j, i, 0)),
            pl.BlockSpec((RB, 1), lambda i, j: (i, 0)),
        ],
        out_shape=[
            jax.ShapeDtypeStruct((4, N_PAD, 16), jnp.float32),
            jax.ShapeDtypeStruct((N, 1), jnp.float32),
        ],
    )(h0, W1, p0, p1)


def _zsum(s, y, dinv, b):
    return pl.pallas_call(
        _zsum_body,
        grid=(GRID_N,),
        in_specs=[
            pl.BlockSpec((4, RB, 16), lambda i: (0, i, 0)),
            pl.BlockSpec((4, RB, 16), lambda i: (0, i, 0)),
            pl.BlockSpec((RB, 1), lambda i: (i, 0)),
            pl.BlockSpec((1, HID), lambda i: (0, 0)),
        ],
        out_specs=[
            pl.BlockSpec((RB, HID), lambda i: (i, 0)),
            pl.BlockSpec((2, HID), lambda i: (0, 0)),
        ],
        out_shape=[
            jax.ShapeDtypeStruct((N, HID), jnp.float32),
            jax.ShapeDtypeStruct((2, HID), jnp.float32),
        ],
        scratch_shapes=[pltpu.VMEM((2, HID), jnp.float32)],
    )(s, y, dinv, b)


def _bnmm(z, sums, g, be, w, dinv):
    return pl.pallas_call(
        _bnmm_body,
        grid=(GRID_N, 4),
        in_specs=[
            pl.BlockSpec((RB, HID), lambda i, j: (i, 0)),
            pl.BlockSpec((2, HID), lambda i, j: (0, 0)),
            pl.BlockSpec((1, HID), lambda i, j: (0, 0)),
            pl.BlockSpec((1, HID), lambda i, j: (0, 0)),
            pl.BlockSpec((1, HID, 16), lambda i, j: (j, 0, 0)),
            pl.BlockSpec((RB, 1), lambda i, j: (i, 0)),
        ],
        out_specs=pl.BlockSpec((1, RB, 16), lambda i, j: (j, i, 0)),
        out_shape=jax.ShapeDtypeStruct((4, N_PAD, 16), jnp.float32),
    )(z, sums, g, be, w, dinv)


def _bnfin(z, sums, g, be):
    return pl.pallas_call(
        _bnfin_body,
        grid=(GRID_N,),
        in_specs=[
            pl.BlockSpec((RB, HID), lambda i: (i, 0)),
            pl.BlockSpec((2, HID), lambda i: (0, 0)),
            pl.BlockSpec((1, HID), lambda i: (0, 0)),
            pl.BlockSpec((1, HID), lambda i: (0, 0)),
        ],
        out_specs=pl.BlockSpec((RB, HID), lambda i: (i, 0)),
        out_shape=jax.ShapeDtypeStruct((N, HID), jnp.float32),
    )(z, sums, g, be)


def _head(hr, WL1, bL1, gL1, beL1, WL3, bL3):
    return pl.pallas_call(
        _head_body,
        grid=(GRID_K,),
        in_specs=[
            pl.BlockSpec((B, KB), lambda i: (0, i)),
            pl.BlockSpec((KB, 128), lambda i: (i, 0)),
            pl.BlockSpec((1, 128), lambda i: (0, 0)),
            pl.BlockSpec((1, 128), lambda i: (0, 0)),
            pl.BlockSpec((1, 128), lambda i: (0, 0)),
            pl.BlockSpec((128, NUM_CLASSES), lambda i: (0, 0)),
            pl.BlockSpec((1, NUM_CLASSES), lambda i: (0, 0)),
        ],
        out_specs=pl.BlockSpec((B, NUM_CLASSES), lambda i: (0, 0)),
        out_shape=jax.ShapeDtypeStruct((B, NUM_CLASSES), jnp.float32),
        scratch_shapes=[pltpu.VMEM((B, 128), jnp.float32)],
    )(hr, WL1, bL1, gL1, beL1, WL3, bL3)


def kernel(x, edge_index, edge_weight, W1, b1, g1, be1, W2, b2, g2, be2,
           W3, b3, g3, be3, WL1, bL1, gL1, beL1, WL3, bL3):
    h0 = jnp.transpose(x, (0, 2, 1)).reshape(N, IN_CH)

    pad = E_PAD - E
    src = jnp.concatenate([edge_index[0], jnp.zeros((pad,), jnp.int32)])
    dst = jnp.concatenate([edge_index[1], jnp.zeros((pad,), jnp.int32)])
    ew = jnp.concatenate([edge_weight, jnp.zeros((pad,), jnp.float32)])
    src_s = src.reshape(NTILE, -1, CHUNK)
    dst_s = dst.reshape(NTILE, -1, CHUNK)
    ew_s = ew.reshape(NTILE, -1, CHUNK)
    dst_d = dst.reshape(NCORE, NTILE, -1, CHUNK)
    ew_d = ew.reshape(NCORE, NTILE, -1, CHUNK)
    zero_q = jnp.zeros((N_PAD, 16), jnp.float32)

    degp = _sc_deg(dst_d, ew_d)
    p0 = degp[:N, None]
    p1 = degp[N_PAD:N_PAD + N, None]

    def wq(w):
        return w.reshape(w.shape[0], 4, 16).transpose(1, 0, 2)

    y, dinv = _mm1(h0, wq(W1), p0, p1)
    W2, W3 = wq(W2), wq(W3)
    params = [(b1, g1, be1, W2), (b2, g2, be2, W3), (b3, g3, be3, None)]
    h = None
    for b, g, be, w_next in params:
        s = _sc_msg(y, src_s, dst_s, ew_s, zero_q)
        z, sums = _zsum(s, y, dinv, b[None, :])
        if w_next is not None:
            y = _bnmm(z, sums, g[None, :], be[None, :], w_next, dinv)
        else:
            h = _bnfin(z, sums, g[None, :], be[None, :])

    hr = h.reshape(B, NPG * HID)
    return _head(hr, WL1, bL1[None, :], gL1[None, :], beL1[None, :],
                 WL3, bL3[None, :])


# speedup vs baseline: 14.5479x; 1.1579x over previous
import functools

import jax
import jax.numpy as jnp
from jax import lax
from jax.experimental import pallas as pl
from jax.experimental.pallas import tpu as pltpu
from jax.experimental.pallas import tpu_sc as plsc

B = 32
NPG = 1516
IN_CH = 128
N = B * NPG
E = 776192
HID = 64
NUM_CLASSES = 10

NTILE = 16
NCORE = 2
N_PAD = 48640
RPT = N_PAD // NTILE
CHUNK = 128
SUP = 8
MSUP = 8
NBUF = 4
LAG = 2
E_PAD = 786432
CPT = (E_PAD // NTILE) // CHUNK
G_S = CPT // MSUP
G_D = (E_PAD // (NTILE * NCORE)) // (CHUNK * SUP)

RB = 3032
GRID_N = N // RB
KPAD = 98304
KB = 16384
GRID_K = KPAD // KB
EPS = 1e-5

_mesh = plsc.VectorSubcoreMesh(core_axis_name="c", subcore_axis_name="s")


@functools.partial(
    pl.kernel,
    mesh=_mesh,
    out_type=jax.ShapeDtypeStruct((NCORE * N_PAD,), jnp.float32),
    scratch_types=[
        pltpu.VMEM_SHARED((N_PAD,), jnp.float32),
        pltpu.VMEM((SUP, CHUNK), jnp.int32),
        pltpu.VMEM((SUP, CHUNK), jnp.float32),
        pltpu.VMEM((RPT,), jnp.float32),
    ],
)
def _sc_deg(dst_hbm, ew_hbm, out_hbm, acc, dst_v, ew_v, zb):
    cid = lax.axis_index("c")
    sid = lax.axis_index("s")

    def zb_body(i, _):
        zb[pl.ds(i * 16, 16)] = jnp.zeros((16,), jnp.float32)
        return 0

    lax.fori_loop(0, RPT // 16, zb_body, 0)
    pltpu.sync_copy(zb, acc.at[pl.ds(sid * RPT, RPT)])
    plsc.subcore_barrier()

    def sup_body(g, _):
        pltpu.sync_copy(dst_hbm.at[cid, sid, pl.ds(g * SUP, SUP)], dst_v)
        pltpu.sync_copy(ew_hbm.at[cid, sid, pl.ds(g * SUP, SUP)], ew_v)

        def chunk_body(j, _):
            pltpu.sync_copy(ew_v.at[j], acc.at[dst_v.at[j]], add=True)
            return 0

        lax.fori_loop(0, SUP, chunk_body, 0)
        return 0

    lax.fori_loop(0, G_D, sup_body, 0)
    plsc.subcore_barrier()
    pltpu.sync_copy(acc.at[pl.ds(sid * RPT, RPT)], zb)
    pltpu.sync_copy(zb, out_hbm.at[pl.ds(cid * N_PAD + sid * RPT, RPT)])


@functools.partial(
    pl.kernel,
    mesh=_mesh,
    compiler_params=pltpu.CompilerParams(use_tc_tiling_on_sc=False),
    out_type=jax.ShapeDtypeStruct((4, N_PAD, 16), jnp.float32),
    scratch_types=[
        pltpu.VMEM_SHARED((N_PAD, 16), jnp.float32),
        pltpu.VMEM_SHARED((N_PAD, 16), jnp.float32),
        pltpu.VMEM((2, MSUP, CHUNK), jnp.int32),
        pltpu.VMEM((2, MSUP, CHUNK), jnp.int32),
        pltpu.VMEM((2, MSUP, CHUNK), jnp.float32),
        pltpu.VMEM((NBUF, CHUNK, 16), jnp.float32),
    ] + [pltpu.SemaphoreType.DMA] * (2 * NBUF + 2),
)
def _sc_msg(yq_hbm, src_hbm, dst_hbm, ew_hbm, zero_hbm, out_hbm,
            acc, table, src_v, dst_v, ew_v, rows, *sems):
    gsem = sems[:NBUF]
    ssem = sems[NBUF:2 * NBUF]
    isem = sems[2 * NBUF:]
    cid = lax.axis_index("c")
    sid = lax.axis_index("s")

    def fire_idx(g, p):
        pltpu.async_copy(src_hbm.at[sid, pl.ds(g * MSUP, MSUP)],
                         src_v.at[p], isem[p])
        pltpu.async_copy(dst_hbm.at[sid, pl.ds(g * MSUP, MSUP)],
                         dst_v.at[p], isem[p])
        pltpu.async_copy(ew_hbm.at[sid, pl.ds(g * MSUP, MSUP)],
                         ew_v.at[p], isem[p])

    def drain_idx(g, p):
        pltpu.make_async_copy(src_hbm.at[sid, pl.ds(g * MSUP, MSUP)],
                              src_v.at[p], isem[p]).wait()
        pltpu.make_async_copy(dst_hbm.at[sid, pl.ds(g * MSUP, MSUP)],
                              dst_v.at[p], isem[p]).wait()
        pltpu.make_async_copy(ew_hbm.at[sid, pl.ds(g * MSUP, MSUP)],
                              ew_v.at[p], isem[p]).wait()

    def do_scale(p, j):
        b = j % NBUF

        def scale_t(t, _):
            k16 = t * 16
            ew16 = ew_v[p, j, pl.ds(k16, 16)]
            for i in range(16):
                w = ew16[i]
                k = k16 + i
                rows[b, k, 0:16] = rows[b, k, 0:16] * w
            return 0

        lax.fori_loop(0, CHUNK // 16, scale_t, 0)

    def pair_body(q, _):
        for p in (0, 1):
            g = 2 * q + p
            drain_idx(g, p)

            @pl.when(g + 1 < G_S)
            def _():
                fire_idx(g + 1, 1 - p)

            gat = [None] * MSUP
            scat = [None] * MSUP

            def stage2(jj):
                bb = jj % NBUF
                gat[jj].wait()
                do_scale(p, jj)
                scat[jj] = pltpu.async_copy(
                    rows.at[bb], acc.at[dst_v.at[p, jj]], ssem[bb], add=True)

            for j in range(MSUP):
                b = j % NBUF
                if j >= NBUF:
                    scat[j - NBUF].wait()
                gat[j] = pltpu.async_copy(table.at[src_v.at[p, j]],
                                          rows.at[b], gsem[b])
                if j >= LAG:
                    stage2(j - LAG)
            for jj in range(MSUP - LAG, MSUP):
                stage2(jj)
            for jj in range(MSUP - NBUF, MSUP):
                scat[jj].wait()
        return 0

    for t in range(2):
        qq = cid * 2 + t
        pltpu.sync_copy(yq_hbm.at[qq, pl.ds(sid * RPT, RPT)],
                        table.at[pl.ds(sid * RPT, RPT)])
        pltpu.sync_copy(zero_hbm.at[pl.ds(sid * RPT, RPT)],
                        acc.at[pl.ds(sid * RPT, RPT)])
        plsc.subcore_barrier()
        fire_idx(0, 0)
        lax.fori_loop(0, G_S // 2, pair_body, 0)
        plsc.subcore_barrier()
        pltpu.sync_copy(acc.at[pl.ds(sid * RPT, RPT)],
                        out_hbm.at[qq, pl.ds(sid * RPT, RPT)])


def _mm1_body(h_ref, w_ref, p0_ref, p1_ref, y_ref, dinv_ref):
    deg = p0_ref[...] + p1_ref[...] + 1.0
    dinv = lax.rsqrt(deg)
    w = jnp.concatenate([w_ref[q] for q in range(4)], axis=1)
    xw = jnp.dot(h_ref[...], w, preferred_element_type=jnp.float32) * dinv
    for q in range(4):
        y_ref[q] = xw[:, 16 * q:16 * (q + 1)]
    dinv_ref[...] = dinv


def _zsum_body(s_ref, y_ref, dinv_ref, b_ref, z_ref, sums_ref, acc):
    i = pl.program_id(0)
    sy = s_ref[...] + y_ref[...]
    z = dinv_ref[...] * jnp.concatenate([sy[q] for q in range(4)],
                                        axis=1) + b_ref[...]
    z_ref[...] = z

    @pl.when(i == 0)
    def _():
        acc[...] = jnp.zeros_like(acc)

    acc[0:1, :] += jnp.sum(z, axis=0, keepdims=True)
    acc[1:2, :] += jnp.sum(z * z, axis=0, keepdims=True)

    @pl.when(i == GRID_N - 1)
    def _():
        sums_ref[...] = acc[...]


def _bn_h(z_ref, sums_ref, g_ref, be_ref):
    mean = sums_ref[0:1, :] * (1.0 / N)
    var = sums_ref[1:2, :] * (1.0 / N) - mean * mean
    inv = lax.rsqrt(var + EPS)
    return jnp.maximum((z_ref[...] - mean) * inv * g_ref[...] + be_ref[...],
                       0.0)


def _bnmm_body(z_ref, sums_ref, g_ref, be_ref, w_ref, dinv_ref, y_ref):
    h = _bn_h(z_ref, sums_ref, g_ref, be_ref)
    w = jnp.concatenate([w_ref[q] for q in range(4)], axis=1)
    y = jnp.dot(h, w, preferred_element_type=jnp.float32) * dinv_ref[...]
    for q in range(4):
        y_ref[q] = y[:, 16 * q:16 * (q + 1)]


def _bnfin_body(z_ref, sums_ref, g_ref, be_ref, h_ref):
    h_ref[...] = _bn_h(z_ref, sums_ref, g_ref, be_ref)


def _head_body(hr_ref, wl1_ref, bl1_ref, gl_ref, bel_ref, wl3_ref, bl3_ref,
               out_ref, acc):
    i = pl.program_id(0)

    @pl.when(i == 0)
    def _():
        acc[...] = jnp.zeros_like(acc)

    acc[...] += jnp.dot(hr_ref[...], wl1_ref[...],
                        preferred_element_type=jnp.float32)

    @pl.when(i == GRID_K - 1)
    def _():
        t = acc[...] + bl1_ref[...]
        m = jnp.mean(t, axis=0, keepdims=True)
        v = jnp.mean(t * t, axis=0, keepdims=True) - m * m
        hh = jnp.maximum(
            (t - m) * lax.rsqrt(v + EPS) * gl_ref[...] + bel_ref[...], 0.0)
        out_ref[...] = jnp.dot(hh, wl3_ref[...],
                               preferred_element_type=jnp.float32) + bl3_ref[...]


def _mm1(h0, W1, p0, p1):
    return pl.pallas_call(
        _mm1_body,
        grid=(GRID_N,),
        in_specs=[
            pl.BlockSpec((RB, IN_CH), lambda i: (i, 0)),
            pl.BlockSpec((4, IN_CH, 16), lambda i: (0, 0, 0)),
            pl.BlockSpec((RB, 1), lambda i: (i, 0)),
            pl.BlockSpec((RB, 1), lambda i: (i, 0)),
        ],
        out_specs=[
            pl.BlockSpec((4, RB, 16), lambda i: (0, i, 0)),
            pl.BlockSpec((RB, 1), lambda i: (i, 0)),
        ],
        out_shape=[
            jax.ShapeDtypeStruct((4, N_PAD, 16), jnp.float32),
            jax.ShapeDtypeStruct((N, 1), jnp.float32),
        ],
    )(h0, W1, p0, p1)


def _zsum(s, y, dinv, b):
    return pl.pallas_call(
        _zsum_body,
        grid=(GRID_N,),
        in_specs=[
            pl.BlockSpec((4, RB, 16), lambda i: (0, i, 0)),
            pl.BlockSpec((4, RB, 16), lambda i: (0, i, 0)),
            pl.BlockSpec((RB, 1), lambda i: (i, 0)),
            pl.BlockSpec((1, HID), lambda i: (0, 0)),
        ],
        out_specs=[
            pl.BlockSpec((RB, HID), lambda i: (i, 0)),
            pl.BlockSpec((2, HID), lambda i: (0, 0)),
        ],
        out_shape=[
            jax.ShapeDtypeStruct((N, HID), jnp.float32),
            jax.ShapeDtypeStruct((2, HID), jnp.float32),
        ],
        scratch_shapes=[pltpu.VMEM((2, HID), jnp.float32)],
    )(s, y, dinv, b)


def _bnmm(z, sums, g, be, w, dinv):
    return pl.pallas_call(
        _bnmm_body,
        grid=(GRID_N,),
        in_specs=[
            pl.BlockSpec((RB, HID), lambda i: (i, 0)),
            pl.BlockSpec((2, HID), lambda i: (0, 0)),
            pl.BlockSpec((1, HID), lambda i: (0, 0)),
            pl.BlockSpec((1, HID), lambda i: (0, 0)),
            pl.BlockSpec((4, HID, 16), lambda i: (0, 0, 0)),
            pl.BlockSpec((RB, 1), lambda i: (i, 0)),
        ],
        out_specs=pl.BlockSpec((4, RB, 16), lambda i: (0, i, 0)),
        out_shape=jax.ShapeDtypeStruct((4, N_PAD, 16), jnp.float32),
    )(z, sums, g, be, w, dinv)


def _bnfin(z, sums, g, be):
    return pl.pallas_call(
        _bnfin_body,
        grid=(GRID_N,),
        in_specs=[
            pl.BlockSpec((RB, HID), lambda i: (i, 0)),
            pl.BlockSpec((2, HID), lambda i: (0, 0)),
            pl.BlockSpec((1, HID), lambda i: (0, 0)),
            pl.BlockSpec((1, HID), lambda i: (0, 0)),
        ],
        out_specs=pl.BlockSpec((RB, HID), lambda i: (i, 0)),
        out_shape=jax.ShapeDtypeStruct((N, HID), jnp.float32),
    )(z, sums, g, be)


def _head(hr, WL1, bL1, gL1, beL1, WL3, bL3):
    return pl.pallas_call(
        _head_body,
        grid=(GRID_K,),
        in_specs=[
            pl.BlockSpec((B, KB), lambda i: (0, i)),
            pl.BlockSpec((KB, 128), lambda i: (i, 0)),
            pl.BlockSpec((1, 128), lambda i: (0, 0)),
            pl.BlockSpec((1, 128), lambda i: (0, 0)),
            pl.BlockSpec((1, 128), lambda i: (0, 0)),
            pl.BlockSpec((128, NUM_CLASSES), lambda i: (0, 0)),
            pl.BlockSpec((1, NUM_CLASSES), lambda i: (0, 0)),
        ],
        out_specs=pl.BlockSpec((B, NUM_CLASSES), lambda i: (0, 0)),
        out_shape=jax.ShapeDtypeStruct((B, NUM_CLASSES), jnp.float32),
        scratch_shapes=[pltpu.VMEM((B, 128), jnp.float32)],
    )(hr, WL1, bL1, gL1, beL1, WL3, bL3)


def kernel(x, edge_index, edge_weight, W1, b1, g1, be1, W2, b2, g2, be2,
           W3, b3, g3, be3, WL1, bL1, gL1, beL1, WL3, bL3):
    h0 = jnp.transpose(x, (0, 2, 1)).reshape(N, IN_CH)

    pad = E_PAD - E
    src = jnp.concatenate([edge_index[0], jnp.zeros((pad,), jnp.int32)])
    dst = jnp.concatenate([edge_index[1], jnp.zeros((pad,), jnp.int32)])
    ew = jnp.concatenate([edge_weight, jnp.zeros((pad,), jnp.float32)])
    src_s = src.reshape(NTILE, -1, CHUNK)
    dst_s = dst.reshape(NTILE, -1, CHUNK)
    ew_s = ew.reshape(NTILE, -1, CHUNK)
    dst_d = dst.reshape(NCORE, NTILE, -1, CHUNK)
    ew_d = ew.reshape(NCORE, NTILE, -1, CHUNK)
    zero_q = jnp.zeros((N_PAD, 16), jnp.float32)

    degp = _sc_deg(dst_d, ew_d)
    p0 = degp[:N, None]
    p1 = degp[N_PAD:N_PAD + N, None]

    def wq(w):
        return w.reshape(w.shape[0], 4, 16).transpose(1, 0, 2)

    y, dinv = _mm1(h0, wq(W1), p0, p1)
    W2, W3 = wq(W2), wq(W3)
    params = [(b1, g1, be1, W2), (b2, g2, be2, W3), (b3, g3, be3, None)]
    h = None
    for b, g, be, w_next in params:
        s = _sc_msg(y, src_s, dst_s, ew_s, zero_q)
        z, sums = _zsum(s, y, dinv, b[None, :])
        if w_next is not None:
            y = _bnmm(z, sums, g[None, :], be[None, :], w_next, dinv)
        else:
            h = _bnfin(z, sums, g[None, :], be[None, :])

    hr = jnp.pad(h.reshape(B, NPG * HID), ((0, 0), (0, KPAD - NPG * HID)))
    WL1p = jnp.pad(WL1, ((0, KPAD - NPG * HID), (0, 0)))
    return _head(hr, WL1p, bL1[None, :], gL1[None, :], beL1[None, :],
                 WL3, bL3[None, :])
